# baseline (speedup 1.0000x reference)
import functools

import jax
import jax.numpy as jnp
from jax import lax
from jax.experimental import pallas as pl
from jax.experimental.pallas import tpu as pltpu
from jax.experimental.pallas import tpu_sc as plsc

_NC = 2
_NS = 16
_NW = _NC * _NS
_CH = 128


@functools.lru_cache(maxsize=None)
def _sc_hist(chunks_per_tile: int, npad: int, n_nodes: int):
    mesh = plsc.VectorSubcoreMesh(core_axis_name="c", subcore_axis_name="s")

    @functools.partial(
        pl.kernel,
        mesh=mesh,
        out_type=[
            jax.ShapeDtypeStruct((_NC, npad), jnp.float32),
            jax.ShapeDtypeStruct((_NC, npad), jnp.float32),
        ],
        scratch_types=[
            pltpu.VMEM((chunks_per_tile * _CH,), jnp.int32),
            pltpu.VMEM((chunks_per_tile * _CH,), jnp.int32),
            pltpu.VMEM((chunks_per_tile * _CH,), jnp.float32),
            pltpu.VMEM_SHARED((npad,), jnp.float32),
            pltpu.VMEM_SHARED((npad,), jnp.float32),
        ],
    )
    def k(dst_hbm, src_hbm, zeros_hbm, deg_out, cnt_out,
          dst_v, src_v, ones_v, deg_sh, cnt_sh):
        c = lax.axis_index("c")
        s = lax.axis_index("s")
        wid = c * _NS + s
        epr = chunks_per_tile * _CH

        @pl.when(s == 0)
        def _zero():
            pltpu.sync_copy(zeros_hbm, deg_sh)
            pltpu.sync_copy(zeros_hbm, cnt_sh)

        base = wid * epr
        pltpu.sync_copy(dst_hbm.at[pl.ds(base, epr)], dst_v)
        pltpu.sync_copy(src_hbm.at[pl.ds(base, epr)], src_v)

        one16 = jnp.full((16,), 1.0, jnp.float32)
        dummy16 = jnp.full((16,), n_nodes, jnp.int32)

        def mkval(j, carry):
            sl = pl.ds(j * 16, 16)
            d = dst_v[sl]
            src_v[sl] = jnp.where(d == 0, src_v[sl], dummy16)
            ones_v[sl] = one16
            return carry

        lax.fori_loop(0, epr // 16, mkval, 0)

        plsc.subcore_barrier()

        pltpu.sync_copy(ones_v, deg_sh.at[dst_v], add=True)
        pltpu.sync_copy(ones_v, cnt_sh.at[src_v], add=True)

        plsc.subcore_barrier()

        @pl.when(s == 0)
        def _out():
            pltpu.sync_copy(deg_sh, deg_out.at[c])
            pltpu.sync_copy(cnt_sh, cnt_out.at[c])

    return k


def _tc_body(x_ref, degp_ref, cntp_ref, h_ref, we_ref, be_ref, wg_ref,
             bg_ref, wih_ref, bih_ref, whh_ref, bhh_ref, out_ref):
    f32 = jnp.float32
    x = x_ref[...]
    enc = jnp.maximum(
        jnp.dot(x, we_ref[...], preferred_element_type=f32) + be_ref[...],
        0.0)

    deg = degp_ref[0:1, :] + degp_ref[1:2, :] + 1.0
    dinv = lax.rsqrt(deg)
    cnt = cntp_ref[0:1, :] + cntp_ref[1:2, :]
    wrow = cnt * dinv

    vsum = jnp.dot(wrow, enc, preferred_element_type=f32)
    dinv0 = dinv[0:1, 0:1]
    v = dinv0 * vsum + (dinv0 * dinv0) * enc[0:1, :]

    agg0 = jnp.dot(v, wg_ref[...], preferred_element_type=f32)
    g = jnp.maximum(agg0 + bg_ref[...], 0.0)

    gi = jnp.dot(g, wih_ref[...], preferred_element_type=f32) + bih_ref[...]
    h0 = h_ref[...]
    gh = jnp.dot(h0, whh_ref[...], preferred_element_type=f32) + bhh_ref[...]
    hdim = h0.shape[1]
    i_r, i_z, i_n = (gi[:, 0:hdim], gi[:, hdim:2 * hdim], gi[:, 2 * hdim:])
    h_r, h_z, h_n = (gh[:, 0:hdim], gh[:, hdim:2 * hdim], gh[:, 2 * hdim:])
    r = jax.nn.sigmoid(i_r + h_r)
    z = jax.nn.sigmoid(i_z + h_z)
    n = jnp.tanh(i_n + r * h_n)
    out_ref[...] = (1.0 - z) * n + z * h0


def kernel(node_features, edge_index, edge_attr, hidden_state,
           W_enc, b_enc, W_gcn, b_gcn, w_ih, b_ih, w_hh, b_hh):
    del edge_attr
    n_nodes, d = node_features.shape
    e = edge_index.shape[1]
    npad = n_nodes + 16

    chunks = -(-e // _CH)
    chunks = -(-chunks // (_NW * 8)) * (_NW * 8)
    e_pad = chunks * _CH
    pad = jnp.full((e_pad - e,), n_nodes, jnp.int32)
    src1d = jnp.concatenate([edge_index[0], pad])
    dst1d = jnp.concatenate([edge_index[1], pad])
    zeros = jnp.zeros((npad,), jnp.float32)

    deg_part, cnt_part = _sc_hist(chunks // _NW, npad, n_nodes)(
        dst1d, src1d, zeros)
    degp = deg_part[:, :n_nodes]
    cntp = cnt_part[:, :n_nodes]

    return pl.pallas_call(
        _tc_body,
        out_shape=jax.ShapeDtypeStruct((1, hidden_state.shape[1]),
                                       jnp.float32),
    )(node_features, degp, cntp, hidden_state,
      W_enc, b_enc.reshape(1, d), W_gcn, b_gcn.reshape(1, d),
      w_ih, b_ih.reshape(1, -1), w_hh, b_hh.reshape(1, -1))


# speedup vs baseline: 1.0079x; 1.0079x over previous
import functools

import jax
import jax.numpy as jnp
from jax import lax
from jax.experimental import pallas as pl
from jax.experimental.pallas import tpu as pltpu
from jax.experimental.pallas import tpu_sc as plsc

_NC = 2
_NS = 16
_NW = _NC * _NS
_CH = 128


@functools.lru_cache(maxsize=None)
def _sc_hist(chunks_per_tile: int, npad: int, n_nodes: int):
    mesh = plsc.VectorSubcoreMesh(core_axis_name="c", subcore_axis_name="s")

    @functools.partial(
        pl.kernel,
        mesh=mesh,
        out_type=[
            jax.ShapeDtypeStruct((_NC, npad), jnp.float32),
            jax.ShapeDtypeStruct((_NC, npad), jnp.float32),
        ],
        scratch_types=[
            pltpu.VMEM((chunks_per_tile, _CH), jnp.int32),
            pltpu.VMEM((chunks_per_tile, _CH), jnp.int32),
            pltpu.VMEM((_CH,), jnp.float32),
            pltpu.VMEM_SHARED((npad,), jnp.float32),
            pltpu.VMEM_SHARED((npad,), jnp.float32),
            pltpu.SemaphoreType.DMA,
        ],
    )
    def k(dst_hbm, src_hbm, zeros_hbm, deg_out, cnt_out,
          dst_v, src_v, ones_v, deg_sh, cnt_sh, sem):
        c = lax.axis_index("c")
        s = lax.axis_index("s")
        wid = c * _NS + s

        @pl.when(s == 0)
        def _zero():
            pltpu.sync_copy(zeros_hbm, deg_sh)
            pltpu.sync_copy(zeros_hbm, cnt_sh)

        base = wid * chunks_per_tile
        pltpu.sync_copy(dst_hbm.at[pl.ds(base, chunks_per_tile)], dst_v)
        pltpu.sync_copy(src_hbm.at[pl.ds(base, chunks_per_tile)], src_v)

        one16 = jnp.full((16,), 1.0, jnp.float32)
        dummy16 = jnp.full((16,), n_nodes, jnp.int32)

        for i in range(_CH // 16):
            ones_v[pl.ds(i * 16, 16)] = one16

        def mkval(j, carry):
            drow = dst_v.at[j]
            srow = src_v.at[j]
            for i in range(_CH // 16):
                sl = pl.ds(i * 16, 16)
                srow[sl] = jnp.where(drow[sl] == 0, srow[sl], dummy16)
            return carry

        lax.fori_loop(0, chunks_per_tile, mkval, 0)

        plsc.subcore_barrier()

        K = 8
        def batch(b, carry):
            handles = []
            for i in range(K):
                j = b * K + i
                handles.append(
                    pltpu.async_copy(ones_v, deg_sh.at[dst_v.at[j]], sem,
                                     add=True))
                handles.append(
                    pltpu.async_copy(ones_v, cnt_sh.at[src_v.at[j]], sem,
                                     add=True))
            for h in handles:
                h.wait()
            return carry

        lax.fori_loop(0, chunks_per_tile // K, batch, 0)

        plsc.subcore_barrier()

        @pl.when(s == 0)
        def _out():
            pltpu.sync_copy(deg_sh, deg_out.at[c])
            pltpu.sync_copy(cnt_sh, cnt_out.at[c])

    return k


def _tc_body(x_ref, degp_ref, cntp_ref, h_ref, we_ref, be_ref, wg_ref,
             bg_ref, wih_ref, bih_ref, whh_ref, bhh_ref, out_ref):
    f32 = jnp.float32
    x = x_ref[...]
    enc = jnp.maximum(
        jnp.dot(x, we_ref[...], preferred_element_type=f32) + be_ref[...],
        0.0)

    deg = degp_ref[0:1, :] + degp_ref[1:2, :] + 1.0
    dinv = lax.rsqrt(deg)
    cnt = cntp_ref[0:1, :] + cntp_ref[1:2, :]
    wrow = cnt * dinv

    vsum = jnp.dot(wrow, enc, preferred_element_type=f32)
    dinv0 = dinv[0:1, 0:1]
    v = dinv0 * vsum + (dinv0 * dinv0) * enc[0:1, :]

    agg0 = jnp.dot(v, wg_ref[...], preferred_element_type=f32)
    g = jnp.maximum(agg0 + bg_ref[...], 0.0)

    gi = jnp.dot(g, wih_ref[...], preferred_element_type=f32) + bih_ref[...]
    h0 = h_ref[...]
    gh = jnp.dot(h0, whh_ref[...], preferred_element_type=f32) + bhh_ref[...]
    hdim = h0.shape[1]
    i_r, i_z, i_n = (gi[:, 0:hdim], gi[:, hdim:2 * hdim], gi[:, 2 * hdim:])
    h_r, h_z, h_n = (gh[:, 0:hdim], gh[:, hdim:2 * hdim], gh[:, 2 * hdim:])
    r = jax.nn.sigmoid(i_r + h_r)
    z = jax.nn.sigmoid(i_z + h_z)
    n = jnp.tanh(i_n + r * h_n)
    out_ref[...] = (1.0 - z) * n + z * h0


def kernel(node_features, edge_index, edge_attr, hidden_state,
           W_enc, b_enc, W_gcn, b_gcn, w_ih, b_ih, w_hh, b_hh):
    del edge_attr
    n_nodes, d = node_features.shape
    e = edge_index.shape[1]
    npad = n_nodes + 16

    chunks = -(-e // _CH)
    chunks = -(-chunks // (_NW * 8)) * (_NW * 8)
    e_pad = chunks * _CH
    pad = jnp.full((e_pad - e,), n_nodes, jnp.int32)
    src1d = jnp.concatenate([edge_index[0], pad]).reshape(chunks, _CH)
    dst1d = jnp.concatenate([edge_index[1], pad]).reshape(chunks, _CH)
    zeros = jnp.zeros((npad,), jnp.float32)

    deg_part, cnt_part = _sc_hist(chunks // _NW, npad, n_nodes)(
        dst1d, src1d, zeros)
    degp = deg_part[:, :n_nodes]
    cntp = cnt_part[:, :n_nodes]

    return pl.pallas_call(
        _tc_body,
        out_shape=jax.ShapeDtypeStruct((1, hidden_state.shape[1]),
                                       jnp.float32),
    )(node_features, degp, cntp, hidden_state,
      W_enc, b_enc.reshape(1, d), W_gcn, b_gcn.reshape(1, d),
      w_ih, b_ih.reshape(1, -1), w_hh, b_hh.reshape(1, -1))


# speedup vs baseline: 3.6615x; 3.6328x over previous
import functools

import jax
import jax.numpy as jnp
from jax import lax
from jax.experimental import pallas as pl
from jax.experimental.pallas import tpu as pltpu
from jax.experimental.pallas import tpu_sc as plsc

_NC = 2
_NS = 16
_NW = _NC * _NS
_CH = 128


@functools.lru_cache(maxsize=None)
def _sc_hist(chunks_per_tile: int, npad: int, n_nodes: int):
    mesh = plsc.VectorSubcoreMesh(core_axis_name="c", subcore_axis_name="s")

    @functools.partial(
        pl.kernel,
        mesh=mesh,
        out_type=[
            jax.ShapeDtypeStruct((_NC, npad), jnp.float32),
            jax.ShapeDtypeStruct((_NC, npad), jnp.float32),
        ],
        scratch_types=[
            pltpu.VMEM((chunks_per_tile, _CH), jnp.int32),
            pltpu.VMEM((chunks_per_tile, _CH), jnp.int32),
            pltpu.VMEM((chunks_per_tile, _CH), jnp.float32),
            pltpu.VMEM((_CH,), jnp.float32),
            pltpu.VMEM_SHARED((npad,), jnp.float32),
            pltpu.VMEM_SHARED((npad,), jnp.float32),
            pltpu.SemaphoreType.DMA,
        ],
    )
    def k(dst_hbm, src_hbm, zeros_hbm, deg_out, cnt_out,
          dst_v, src_v, val_v, ones_v, deg_sh, cnt_sh, sem):
        c = lax.axis_index("c")
        s = lax.axis_index("s")
        wid = c * _NS + s

        @pl.when(s == 0)
        def _zero():
            pltpu.sync_copy(zeros_hbm, deg_sh)
            pltpu.sync_copy(zeros_hbm, cnt_sh)

        base = wid * chunks_per_tile
        pltpu.sync_copy(dst_hbm.at[pl.ds(base, chunks_per_tile)], dst_v)
        pltpu.sync_copy(src_hbm.at[pl.ds(base, chunks_per_tile)], src_v)

        one16 = jnp.full((16,), 1.0, jnp.float32)
        zero16 = jnp.zeros((16,), jnp.float32)

        for i in range(_CH // 16):
            ones_v[pl.ds(i * 16, 16)] = one16

        def mkval(j, carry):
            drow = dst_v.at[j]
            vrow = val_v.at[j]
            for i in range(_CH // 16):
                sl = pl.ds(i * 16, 16)
                vrow[sl] = jnp.where(drow[sl] == 0, one16, zero16)
            return carry

        lax.fori_loop(0, chunks_per_tile, mkval, 0)

        plsc.subcore_barrier()

        K = 8
        def batch(b, carry):
            handles = []
            for i in range(K):
                j = b * K + i
                handles.append(
                    pltpu.async_copy(ones_v, deg_sh.at[dst_v.at[j]], sem,
                                     add=True))
                handles.append(
                    pltpu.async_copy(val_v.at[j], cnt_sh.at[src_v.at[j]], sem,
                                     add=True))
            for h in handles:
                h.wait()
            return carry

        lax.fori_loop(0, chunks_per_tile // K, batch, 0)

        plsc.subcore_barrier()

        @pl.when(s == 0)
        def _out():
            pltpu.sync_copy(deg_sh, deg_out.at[c])
            pltpu.sync_copy(cnt_sh, cnt_out.at[c])

    return k


def _tc_body(x_ref, degp_ref, cntp_ref, h_ref, we_ref, be_ref, wg_ref,
             bg_ref, wih_ref, bih_ref, whh_ref, bhh_ref, out_ref):
    f32 = jnp.float32
    x = x_ref[...]
    enc = jnp.maximum(
        jnp.dot(x, we_ref[...], preferred_element_type=f32) + be_ref[...],
        0.0)

    deg = degp_ref[0:1, :] + degp_ref[1:2, :] + 1.0
    dinv = lax.rsqrt(deg)
    cnt = cntp_ref[0:1, :] + cntp_ref[1:2, :]
    wrow = cnt * dinv

    vsum = jnp.dot(wrow, enc, preferred_element_type=f32)
    dinv0 = dinv[0:1, 0:1]
    v = dinv0 * vsum + (dinv0 * dinv0) * enc[0:1, :]

    agg0 = jnp.dot(v, wg_ref[...], preferred_element_type=f32)
    g = jnp.maximum(agg0 + bg_ref[...], 0.0)

    gi = jnp.dot(g, wih_ref[...], preferred_element_type=f32) + bih_ref[...]
    h0 = h_ref[...]
    gh = jnp.dot(h0, whh_ref[...], preferred_element_type=f32) + bhh_ref[...]
    hdim = h0.shape[1]
    i_r, i_z, i_n = (gi[:, 0:hdim], gi[:, hdim:2 * hdim], gi[:, 2 * hdim:])
    h_r, h_z, h_n = (gh[:, 0:hdim], gh[:, hdim:2 * hdim], gh[:, 2 * hdim:])
    r = jax.nn.sigmoid(i_r + h_r)
    z = jax.nn.sigmoid(i_z + h_z)
    n = jnp.tanh(i_n + r * h_n)
    out_ref[...] = (1.0 - z) * n + z * h0


def kernel(node_features, edge_index, edge_attr, hidden_state,
           W_enc, b_enc, W_gcn, b_gcn, w_ih, b_ih, w_hh, b_hh):
    del edge_attr
    n_nodes, d = node_features.shape
    e = edge_index.shape[1]
    npad = n_nodes + 16

    chunks = -(-e // _CH)
    chunks = -(-chunks // (_NW * 8)) * (_NW * 8)
    e_pad = chunks * _CH
    pad = jnp.full((e_pad - e,), n_nodes, jnp.int32)
    src1d = jnp.concatenate([edge_index[0], pad]).reshape(chunks, _CH)
    dst1d = jnp.concatenate([edge_index[1], pad]).reshape(chunks, _CH)
    zeros = jnp.zeros((npad,), jnp.float32)

    deg_part, cnt_part = _sc_hist(chunks // _NW, npad, n_nodes)(
        dst1d, src1d, zeros)
    degp = deg_part[:, :n_nodes]
    cntp = cnt_part[:, :n_nodes]

    return pl.pallas_call(
        _tc_body,
        out_shape=jax.ShapeDtypeStruct((1, hidden_state.shape[1]),
                                       jnp.float32),
    )(node_features, degp, cntp, hidden_state,
      W_enc, b_enc.reshape(1, d), W_gcn, b_gcn.reshape(1, d),
      w_ih, b_ih.reshape(1, -1), w_hh, b_hh.reshape(1, -1))


# speedup vs baseline: 5.8162x; 1.5885x over previous
import functools

import jax
import jax.numpy as jnp
from jax import lax
from jax.experimental import pallas as pl
from jax.experimental.pallas import tpu as pltpu
from jax.experimental.pallas import tpu_sc as plsc

_NC = 2
_NS = 16
_NW = _NC * _NS


@functools.lru_cache(maxsize=None)
def _sc_hist(n_edges: int, n_bins: int):
    ept = n_edges // _NW
    assert ept * _NW == n_edges and ept % 16 == 0 and ept % 8 == 0
    mesh = plsc.VectorSubcoreMesh(core_axis_name="c", subcore_axis_name="s")

    @functools.partial(
        pl.kernel,
        mesh=mesh,
        out_type=[
            jax.ShapeDtypeStruct((_NC, n_bins), jnp.float32),
            jax.ShapeDtypeStruct((_NC, n_bins), jnp.float32),
        ],
        scratch_types=[
            pltpu.VMEM((ept,), jnp.int32),
            pltpu.VMEM((ept,), jnp.int32),
            pltpu.VMEM((ept,), jnp.float32),
            pltpu.VMEM((ept,), jnp.float32),
            pltpu.VMEM_SHARED((n_bins,), jnp.float32),
            pltpu.VMEM_SHARED((n_bins,), jnp.float32),
        ],
    )
    def k(ei_hbm, zeros_hbm, deg_out, cnt_out,
          dst_v, src_v, val_v, ones_v, deg_sh, cnt_sh):
        c = lax.axis_index("c")
        s = lax.axis_index("s")
        wid = c * _NS + s

        @pl.when(s == 0)
        def _zero():
            pltpu.sync_copy(zeros_hbm, deg_sh)
            pltpu.sync_copy(zeros_hbm, cnt_sh)

        pltpu.sync_copy(ei_hbm.at[pl.ds(n_edges + wid * ept, ept)], dst_v)
        pltpu.sync_copy(ei_hbm.at[pl.ds(wid * ept, ept)], src_v)

        one16 = jnp.full((16,), 1.0, jnp.float32)
        zero16 = jnp.zeros((16,), jnp.float32)

        def mkval(j, carry):
            sl = pl.ds(j * 16, 16)
            val_v[sl] = jnp.where(dst_v[sl] == 0, one16, zero16)
            ones_v[sl] = one16
            return carry

        lax.fori_loop(0, ept // 16, mkval, 0)

        plsc.subcore_barrier()

        pltpu.sync_copy(ones_v, deg_sh.at[dst_v], add=True)
        pltpu.sync_copy(val_v, cnt_sh.at[src_v], add=True)

        plsc.subcore_barrier()

        @pl.when(s == 0)
        def _out():
            pltpu.sync_copy(deg_sh, deg_out.at[c])
            pltpu.sync_copy(cnt_sh, cnt_out.at[c])

    return k


def _tc_body(x_ref, degp_ref, cntp_ref, h_ref, we_ref, be_ref, wg_ref,
             bg_ref, wih_ref, bih_ref, whh_ref, bhh_ref, out_ref):
    f32 = jnp.float32
    x = x_ref[...]
    enc = jnp.maximum(
        jnp.dot(x, we_ref[...], preferred_element_type=f32) + be_ref[...],
        0.0)

    deg = degp_ref[0:1, :] + degp_ref[1:2, :] + 1.0
    dinv = lax.rsqrt(deg)
    cnt = cntp_ref[0:1, :] + cntp_ref[1:2, :]
    wrow = cnt * dinv

    vsum = jnp.dot(wrow, enc, preferred_element_type=f32)
    dinv0 = dinv[0:1, 0:1]
    v = dinv0 * vsum + (dinv0 * dinv0) * enc[0:1, :]

    agg0 = jnp.dot(v, wg_ref[...], preferred_element_type=f32)
    g = jnp.maximum(agg0 + bg_ref[...], 0.0)

    gi = jnp.dot(g, wih_ref[...], preferred_element_type=f32) + bih_ref[...]
    h0 = h_ref[...]
    gh = jnp.dot(h0, whh_ref[...], preferred_element_type=f32) + bhh_ref[...]
    hdim = h0.shape[1]
    i_r, i_z, i_n = (gi[:, 0:hdim], gi[:, hdim:2 * hdim], gi[:, 2 * hdim:])
    h_r, h_z, h_n = (gh[:, 0:hdim], gh[:, hdim:2 * hdim], gh[:, 2 * hdim:])
    r = jax.nn.sigmoid(i_r + h_r)
    z = jax.nn.sigmoid(i_z + h_z)
    n = jnp.tanh(i_n + r * h_n)
    out_ref[...] = (1.0 - z) * n + z * h0


def kernel(node_features, edge_index, edge_attr, hidden_state,
           W_enc, b_enc, W_gcn, b_gcn, w_ih, b_ih, w_hh, b_hh):
    del edge_attr
    n_nodes, d = node_features.shape
    e = edge_index.shape[1]

    ei_flat = edge_index.reshape(2 * e)
    zeros = jnp.zeros((n_nodes,), jnp.float32)

    degp, cntp = _sc_hist(e, n_nodes)(ei_flat, zeros)

    return pl.pallas_call(
        _tc_body,
        out_shape=jax.ShapeDtypeStruct((1, hidden_state.shape[1]),
                                       jnp.float32),
    )(node_features, degp, cntp, hidden_state,
      W_enc, b_enc.reshape(1, d), W_gcn, b_gcn.reshape(1, d),
      w_ih, b_ih.reshape(1, -1), w_hh, b_hh.reshape(1, -1))
